# baseline (speedup 1.0000x reference)
import functools

import jax
import jax.numpy as jnp
from jax import lax
from jax.experimental import pallas as pl
from jax.experimental.pallas import tpu as pltpu
from jax.experimental.pallas import tpu_sc as plsc

_NSAMPLE = 1_000_000
_NREP = 64
_BATCH = 16384

_info = plsc.get_sparse_core_info()
_NC, _NS = _info.num_cores, _info.num_subcores
_NW = _NC * _NS
_B_PER_W = _BATCH // _NW


@functools.partial(
    pl.kernel,
    mesh=plsc.VectorSubcoreMesh(core_axis_name="c", subcore_axis_name="s"),
    out_type=jax.ShapeDtypeStruct((_BATCH, _NREP), jnp.float32),
    scratch_types=[
        pltpu.VMEM((_B_PER_W,), jnp.int32),
        pltpu.VMEM((_B_PER_W, _NREP), jnp.float32),
        pltpu.SemaphoreType.DMA,
    ],
    compiler_params=pltpu.CompilerParams(use_tc_tiling_on_sc=False),
)
def _gather_kernel(idx_hbm, z_hbm, out_hbm, idx_v, rows_v, sem):
    wid = lax.axis_index("s") * _NC + lax.axis_index("c")
    base = wid * _B_PER_W
    pltpu.sync_copy(idx_hbm.at[pl.ds(base, _B_PER_W)], idx_v)
    pltpu.async_copy(z_hbm.at[idx_v], rows_v, sem).wait()
    pltpu.sync_copy(rows_v, out_hbm.at[pl.ds(base, _B_PER_W)])


def kernel(idx, z):
    return _gather_kernel(idx.astype(jnp.int32), z)


# speedup vs baseline: 1.7319x; 1.7319x over previous
import functools

import jax
import jax.numpy as jnp
from jax import lax
from jax.experimental import pallas as pl
from jax.experimental.pallas import tpu as pltpu
from jax.experimental.pallas import tpu_sc as plsc

_NSAMPLE = 1_000_000
_NREP = 64
_BATCH = 16384

_info = plsc.get_sparse_core_info()
_NC, _NS = _info.num_cores, _info.num_subcores
_NW = _NC * _NS
_B_PER_W = _BATCH // _NW


@functools.partial(
    pl.kernel,
    mesh=plsc.VectorSubcoreMesh(core_axis_name="c", subcore_axis_name="s"),
    out_type=jax.ShapeDtypeStruct((_BATCH, _NREP), jnp.float32),
    scratch_types=[
        pltpu.VMEM((_B_PER_W,), jnp.int32),
        pltpu.VMEM((_B_PER_W, _NREP), jnp.float32),
        pltpu.SemaphoreType.DMA,
    ],
)
def _gather_kernel(idx_hbm, z_hbm, out_hbm, idx_v, rows_v, sem):
    wid = lax.axis_index("s") * _NC + lax.axis_index("c")
    base = wid * _B_PER_W
    pltpu.sync_copy(idx_hbm.at[pl.ds(base, _B_PER_W)], idx_v)

    def grp_body(g, _):
        vec = idx_v[pl.ds(g * 16, 16)]
        for l in range(16):
            t = vec[l]
            pltpu.async_copy(z_hbm.at[pl.ds(t, 1)],
                             rows_v.at[pl.ds(g * 16 + l, 1)], sem)
        return 0

    lax.fori_loop(0, _B_PER_W // 16, grp_body, 0)

    def drain_body(k, _):
        pltpu.make_async_copy(z_hbm.at[pl.ds(0, 1)], rows_v.at[pl.ds(0, 1)],
                              sem).wait()
        return 0

    lax.fori_loop(0, _B_PER_W, drain_body, 0)

    pltpu.sync_copy(rows_v, out_hbm.at[pl.ds(base, _B_PER_W)])


def kernel(idx, z):
    return _gather_kernel(idx.astype(jnp.int32), z)


# speedup vs baseline: 1.7328x; 1.0005x over previous
import functools

import jax
import jax.numpy as jnp
from jax import lax
from jax.experimental import pallas as pl
from jax.experimental.pallas import tpu as pltpu
from jax.experimental.pallas import tpu_sc as plsc

_NSAMPLE = 1_000_000
_NREP = 64
_BATCH = 16384

_info = plsc.get_sparse_core_info()
_NC, _NS = _info.num_cores, _info.num_subcores
_NW = _NC * _NS
_B_PER_W = _BATCH // _NW


@functools.partial(
    pl.kernel,
    mesh=plsc.VectorSubcoreMesh(core_axis_name="c", subcore_axis_name="s"),
    out_type=jax.ShapeDtypeStruct((_BATCH, _NREP), jnp.float32),
    scratch_types=[
        pltpu.VMEM((_B_PER_W,), jnp.int32),
        pltpu.VMEM((_B_PER_W, _NREP), jnp.float32),
        pltpu.SemaphoreType.DMA,
    ],
)
def _gather_kernel(idx_hbm, z_hbm, out_hbm, idx_v, rows_v, sem):
    wid = lax.axis_index("s") * _NC + lax.axis_index("c")
    base = wid * _B_PER_W
    pltpu.sync_copy(idx_hbm.at[pl.ds(base, _B_PER_W)], idx_v)

    with jax.named_scope("issue_phase"):
        def grp_body(g, _):
            vec = idx_v[pl.ds(g * 16, 16)]
            for l in range(16):
                t = vec[l]
                pltpu.async_copy(z_hbm.at[pl.ds(t, 1)],
                                 rows_v.at[pl.ds(g * 16 + l, 1)], sem)
            return 0

        lax.fori_loop(0, _B_PER_W // 16, grp_body, 0)

    with jax.named_scope("drain_phase"):
        def drain_body(k, _):
            pltpu.make_async_copy(z_hbm.at[pl.ds(0, 1)],
                                  rows_v.at[pl.ds(0, 1)], sem).wait()
            return 0

        lax.fori_loop(0, _B_PER_W, drain_body, 0)

    with jax.named_scope("writeback_phase"):
        pltpu.sync_copy(rows_v, out_hbm.at[pl.ds(base, _B_PER_W)])


def kernel(idx, z):
    return _gather_kernel(idx.astype(jnp.int32), z)
